# Initial kernel scaffold; baseline (speedup 1.0000x reference)
import jax
import jax.numpy as jnp
from jax.experimental import pallas as pl


def kernel(feature, shuf, edge_index, edge_weight, sparse, msk, samp_bias1, samp_bias2, W_gcn, W_bil, b_bil, H_emb):
    raise NotImplementedError("write your pallas kernel here")


# speedup vs baseline: 4.3335x; 4.3335x over previous
import functools

import jax
import jax.numpy as jnp
from jax import lax
from jax.experimental import pallas as pl
from jax.experimental.pallas import tpu as pltpu
from jax.experimental.pallas import tpu_sc as plsc

G = 2
N = 10000
F = 128
H = 128
E = 320000

NC = 2
NS = 16
LANES = 16

CH = 128
TOT_CH = E // CH
RPT = 624
ZR = 208

BN = 1000
NB = N // BN


def _mm_body(feat_ref, shuf_ref, w_ref, out_ref):
    w = w_ref[0]
    out_ref[0, 0] = jnp.dot(feat_ref[0, 0], w, preferred_element_type=jnp.float32)
    out_ref[1, 0] = jnp.dot(shuf_ref[0, 0], w, preferred_element_type=jnp.float32)


def _seq_tables(feature, shuf, W_gcn):
    out = pl.pallas_call(
        _mm_body,
        grid=(G, NB),
        in_specs=[
            pl.BlockSpec((1, 1, BN, F), lambda g, j: (g, 0, j, 0)),
            pl.BlockSpec((1, 1, BN, F), lambda g, j: (g, 0, j, 0)),
            pl.BlockSpec((1, F, H), lambda g, j: (g, 0, 0)),
        ],
        out_specs=pl.BlockSpec((2, 1, BN, H), lambda g, j: (0, g, j, 0)),
        out_shape=jax.ShapeDtypeStruct((2, G, N, H), jnp.float32),
    )(feature, shuf, W_gcn)
    return out.reshape(2 * G, N, H)


def _sc_body(seq_hbm, row_hbm, col_hbm, w_hbm, agg_hbm,
             acc, colv, rowv, wv, gv, zv, sem):
    c = lax.axis_index("c")
    s = lax.axis_index("s")

    def _zrow(r, carry):
        for k in range(H // LANES):
            zv[r, pl.ds(k * LANES, LANES)] = jnp.zeros((LANES,), jnp.float32)
        return carry
    lax.fori_loop(0, ZR, _zrow, 0)

    nch = 156 + (s < 4).astype(jnp.int32)

    base_r = pl.multiple_of(s * RPT, 8)
    for g in range(G):
        t = c * G + g
        for z in range(3):
            pltpu.sync_copy(zv, acc.at[pl.ds(base_r + z * ZR, ZR), :])

        @pl.when(s == NS - 1)
        def _():
            pltpu.sync_copy(zv.at[pl.ds(0, 16), :],
                            acc.at[pl.ds(N - 16, 16), :])
        plsc.subcore_barrier()

        def _chunk(tt, carry):
            off = (s + NS * tt) * CH
            pltpu.sync_copy(col_hbm.at[g, pl.ds(off, CH)], colv)
            pltpu.sync_copy(row_hbm.at[g, pl.ds(off, CH)], rowv)
            pltpu.sync_copy(w_hbm.at[g, pl.ds(off, CH)], wv)
            pltpu.async_copy(seq_hbm.at[t].at[colv], gv, sem).wait()
            def _grp(q, carry2):
                w16 = wv[pl.ds(q * LANES, LANES)]
                for e in range(LANES):
                    wb = lax.gather(
                        w16, jnp.full((LANES, 1), e, jnp.int32),
                        lax.GatherDimensionNumbers(
                            offset_dims=(), collapsed_slice_dims=(0,),
                            start_index_map=(0,)),
                        slice_sizes=(1,),
                        mode=lax.GatherScatterMode.PROMISE_IN_BOUNDS)
                    r = q * LANES + e
                    for k in range(H // LANES):
                        sl = pl.ds(k * LANES, LANES)
                        gv[r, sl] = gv[r, sl] * wb
                return carry2
            lax.fori_loop(0, CH // LANES, _grp, 0)
            pltpu.sync_copy(gv, acc.at[rowv], add=True)
            return carry
        lax.fori_loop(0, nch, _chunk, 0)

        plsc.subcore_barrier()
        pltpu.sync_copy(acc.at[pl.ds(base_r, RPT), :],
                        agg_hbm.at[t, pl.ds(base_r, RPT), :])

        @pl.when(s == NS - 1)
        def _():
            pltpu.sync_copy(acc.at[pl.ds(N - 16, 16), :],
                            agg_hbm.at[t, pl.ds(N - 16, 16), :])
        plsc.subcore_barrier()


def _sc_spmm(seq, row, col, w):
    mesh = plsc.VectorSubcoreMesh(core_axis_name="c", subcore_axis_name="s")
    kern = pl.kernel(
        _sc_body,
        out_type=jax.ShapeDtypeStruct((2 * G, N, H), jnp.float32),
        mesh=mesh,
        scratch_types=[
            pltpu.VMEM_SHARED((N, H), jnp.float32),
            pltpu.VMEM((CH,), jnp.int32),
            pltpu.VMEM((CH,), jnp.int32),
            pltpu.VMEM((CH,), jnp.float32),
            pltpu.VMEM((CH, H), jnp.float32),
            pltpu.VMEM((ZR, H), jnp.float32),
            pltpu.SemaphoreType.DMA,
        ],
    )
    return kern(seq, row, col, w)


def _sums_body(agg_ref, out_ref):
    j = pl.program_id(1)
    part = jnp.sum(jnp.maximum(agg_ref[0], 0.0), axis=0, keepdims=True)
    part8 = jnp.broadcast_to(part, (8, H))

    @pl.when(j == 0)
    def _():
        out_ref[0] = part8

    @pl.when(j > 0)
    def _():
        out_ref[0] = out_ref[0] + part8


def _node_sums(agg):
    return pl.pallas_call(
        _sums_body,
        grid=(2 * G, NB),
        in_specs=[pl.BlockSpec((1, BN, H), lambda t, j: (t, j, 0))],
        out_specs=pl.BlockSpec((1, 8, H), lambda t, j: (t, 0, 0)),
        out_shape=jax.ShapeDtypeStruct((2 * G, 8, H), jnp.float32),
    )(agg)


def _fin_body(agg_ref, hemb_ref, sums_ref, wbil_ref, sb1_ref, sb2_ref,
              bbil_ref, sc_ref, reg_ref):
    j = pl.program_id(0)
    b = bbil_ref[0, 0]
    cvec = jax.nn.sigmoid(sums_ref[0:2, 0, :] / float(N))
    v = jnp.einsum("gk,hk->gh", cvec, wbil_ref[...],
                   preferred_element_type=jnp.float32)
    xs = []
    for t in range(2 * G):
        g = t % G
        x = jnp.maximum(agg_ref[t], 0.0)
        xs.append(x)
        s = jnp.dot(x, v[g][:, None],
                    preferred_element_type=jnp.float32)
        bias = sb1_ref[...] if t < G else sb2_ref[...]
        sc_ref[t] = s + b + bias

    h1a = (xs[0] + xs[1]) * 0.5
    h2a = (xs[2] + xs[3]) * 0.5
    d1 = hemb_ref[0] - h1a
    d2 = hemb_ref[0] - h2a
    p = jnp.broadcast_to(jnp.sum(d1 * d1, axis=0, keepdims=True), (8, H))
    q = jnp.broadcast_to(jnp.sum(d2 * d2, axis=0, keepdims=True), (8, H))

    @pl.when(j == 0)
    def _():
        reg_ref[0] = p
        reg_ref[1] = q

    @pl.when(j > 0)
    def _():
        reg_ref[0] = reg_ref[0] + p
        reg_ref[1] = reg_ref[1] + q


def _finish(agg, H_emb, sums, W_bil, samp_bias1, samp_bias2, b_bil):
    return pl.pallas_call(
        _fin_body,
        grid=(NB,),
        in_specs=[
            pl.BlockSpec((2 * G, BN, H), lambda j: (0, j, 0)),
            pl.BlockSpec((1, BN, H), lambda j: (0, j, 0)),
            pl.BlockSpec((2 * G, 8, H), lambda j: (0, 0, 0)),
            pl.BlockSpec((H, H), lambda j: (0, 0)),
            pl.BlockSpec((BN, 1), lambda j: (j, 0)),
            pl.BlockSpec((BN, 1), lambda j: (j, 0)),
            pl.BlockSpec(memory_space=pltpu.SMEM),
        ],
        out_specs=[
            pl.BlockSpec((2 * G, BN, 1), lambda j: (0, j, 0)),
            pl.BlockSpec((2, 8, H), lambda j: (0, 0, 0)),
        ],
        out_shape=[
            jax.ShapeDtypeStruct((2 * G, N, 1), jnp.float32),
            jax.ShapeDtypeStruct((2, 8, H), jnp.float32),
        ],
    )(agg, H_emb, sums, W_bil, samp_bias1.reshape(N, 1),
      samp_bias2.reshape(N, 1), b_bil)


def kernel(feature, shuf, edge_index, edge_weight, sparse, msk,
           samp_bias1, samp_bias2, W_gcn, W_bil, b_bil, H_emb):
    seq = _seq_tables(feature, shuf, W_gcn)
    row = edge_index[:, 0, :]
    col = edge_index[:, 1, :]
    agg = _sc_spmm(seq, row, col, edge_weight)
    sums = _node_sums(agg)
    sc, regp = _finish(agg, H_emb, sums, W_bil, samp_bias1, samp_bias2,
                       b_bil.reshape(1, 1))
    sc = sc[..., 0]
    logits = jnp.stack([
        jnp.concatenate([sc[0], sc[2]], axis=0)[None, :],
        jnp.concatenate([sc[1], sc[3]], axis=0)[None, :],
    ], axis=0)
    reg_loss = jnp.sum(regp[0, 0]) - jnp.sum(regp[1, 0])
    return (logits, reg_loss)
